# Initial kernel scaffold; baseline (speedup 1.0000x reference)
import jax
import jax.numpy as jnp
from jax.experimental import pallas as pl


def kernel(h, targets, x0, t, Ws, bs, Wc, bc, W1, b1, W2, b2, W3, b3):
    raise NotImplementedError("write your pallas kernel here")


# speedup vs baseline: 1.6935x; 1.6935x over previous
import functools

import jax
import jax.numpy as jnp
from jax.experimental import pallas as pl
from jax.experimental.pallas import tpu as pltpu

N, H = 65536, 1024
BLK = 512


def _body(h_ref, tg_ref, x0_ref, t_ref, wsc_ref, w1h_ref, w1r_ref,
          b1_ref, w2_ref, b2_ref, w3p_ref, scal_ref, out_ref):
    bs = scal_ref[0]
    bc = scal_ref[1]
    wc_last = scal_ref[2]
    b3 = scal_ref[3]

    h = h_ref[...]
    sc = jnp.dot(h, wsc_ref[...], preferred_element_type=jnp.float32)
    s_logits = sc[:, 0:1] + bs
    c_pre = sc[:, 1:2] + s_logits * wc_last + bc
    c_lambda = (jnp.maximum(c_pre, 0.0)
                + jnp.log1p(jnp.exp(-jnp.abs(c_pre))) + 1e-6)

    tg = tg_ref[...]
    x0 = x0_ref[...]
    tv = t_ref[...]
    mask = (tg > 0.0).astype(jnp.float32)
    y = jnp.log1p(jnp.maximum(tg, 0.0))
    x_t = (1.0 - tv) * x0 + tv * y
    target_v = y - x0

    z1p = (jnp.dot(h, w1h_ref[...], preferred_element_type=jnp.float32)
           + c_lambda * w1r_ref[0:1, :]
           + x_t * w1r_ref[1:2, :]
           + tv * w1r_ref[2:3, :]
           + b1_ref[...])
    z1 = z1p * jax.nn.sigmoid(z1p)
    z2p = jnp.dot(z1, w2_ref[...], preferred_element_type=jnp.float32) + b2_ref[...]
    z2 = z2p * jax.nn.sigmoid(z2p)
    pv = jnp.dot(z2, w3p_ref[...], preferred_element_type=jnp.float32)[:, 0:1] + b3

    r = pv - target_v
    main_v = mask * r * r
    bce_v = (jnp.maximum(s_logits, 0.0) - s_logits * mask
             + jnp.log1p(jnp.exp(-jnp.abs(s_logits))))
    c_d = c_lambda - y
    c_v = c_d * c_d

    lane = jax.lax.broadcasted_iota(jnp.int32, (h.shape[0], 128), 1)
    acc = (jnp.where(lane == 0, bce_v, 0.0)
           + jnp.where(lane == 1, c_v, 0.0)
           + jnp.where(lane == 2, main_v, 0.0)
           + jnp.where(lane == 3, mask, 0.0))
    out_ref[...] = jnp.sum(acc, axis=0, keepdims=True)[None]


@jax.jit
def kernel(h, targets, x0, t, Ws, bs, Wc, bc, W1, b1, W2, b2, W3, b3):
    f32 = jnp.float32
    wsc = jnp.zeros((H, 128), f32).at[:, 0].set(Ws).at[:, 1].set(Wc[:H])
    w3p = jnp.zeros((H, 128), f32).at[:, 0].set(W3)
    scal = jnp.stack([bs, bc, Wc[H], b3]).astype(f32)

    nblk = N // BLK
    parts = pl.pallas_call(
        _body,
        grid=(nblk,),
        in_specs=[
            pl.BlockSpec((BLK, H), lambda i: (i, 0)),
            pl.BlockSpec((BLK, 1), lambda i: (i, 0)),
            pl.BlockSpec((BLK, 1), lambda i: (i, 0)),
            pl.BlockSpec((BLK, 1), lambda i: (i, 0)),
            pl.BlockSpec((H, 128), lambda i: (0, 0)),
            pl.BlockSpec((H, H), lambda i: (0, 0)),
            pl.BlockSpec((3, H), lambda i: (0, 0)),
            pl.BlockSpec((1, H), lambda i: (0, 0)),
            pl.BlockSpec((H, H), lambda i: (0, 0)),
            pl.BlockSpec((1, H), lambda i: (0, 0)),
            pl.BlockSpec((H, 128), lambda i: (0, 0)),
            pl.BlockSpec(memory_space=pltpu.SMEM),
        ],
        out_specs=pl.BlockSpec((1, 1, 128), lambda i: (i, 0, 0)),
        out_shape=jax.ShapeDtypeStruct((nblk, 1, 128), f32),
        compiler_params=pltpu.CompilerParams(
            dimension_semantics=("parallel",),
            vmem_limit_bytes=100 * 1024 * 1024,
        ),
    )(h, targets[:, None], x0[:, None], t[:, None],
      wsc, W1[:H], W1[H:], b1[None, :], W2, b2[None, :], w3p, scal)

    sums = jnp.sum(parts[:, 0, :], axis=0)
    s_loss = sums[0] / N
    c_loss = sums[1] / N
    main_loss = sums[2] / jnp.maximum(sums[3], 1.0)
    total = main_loss + 0.05 * s_loss + 0.05 * c_loss
    return jnp.stack([main_loss, s_loss, c_loss, total])
